# baseline (speedup 1.0000x reference)
import functools

import jax
import jax.numpy as jnp
from jax import lax
from jax.experimental import pallas as pl
from jax.experimental.pallas import tpu as pltpu
from jax.experimental.pallas import tpu_sc as plsc

N_JRNL = 1000000
JRNL_DIM = 32
MESH_SIZE = 28340
HIDDEN_SIZE = 128
N_PROBES = 3
BATCH = 1024
CTX_DIM = HIDDEN_SIZE * N_PROBES
IN_FEAT = CTX_DIM + JRNL_DIM

_NC = 2
_NS = 16
_NW = _NC * _NS
_B_PER_W = BATCH // _NW


@functools.cache
def _get_sc_gather():
    @functools.partial(
        pl.kernel,
        out_type=jax.ShapeDtypeStruct((BATCH, JRNL_DIM), jnp.float32),
        mesh=plsc.VectorSubcoreMesh(
            core_axis_name="c", subcore_axis_name="s",
            num_cores=_NC, num_subcores=_NS),
        scratch_types=[
            pltpu.VMEM((_B_PER_W,), jnp.int32),
            pltpu.VMEM((_B_PER_W, JRNL_DIM), jnp.float32),
            pltpu.SemaphoreType.DMA,
        ],
        compiler_params=pltpu.CompilerParams(use_tc_tiling_on_sc=False),
    )
    def _sc_gather(idx_hbm, table_hbm, out_hbm, idx_v, rows_v, sem):
        wid = lax.axis_index("s") * _NC + lax.axis_index("c")
        base = wid * _B_PER_W
        pltpu.sync_copy(idx_hbm.at[pl.ds(base, _B_PER_W)], idx_v)
        pltpu.async_copy(table_hbm.at[idx_v], rows_v, sem).wait()
        pltpu.sync_copy(rows_v, out_hbm.at[pl.ds(base, _B_PER_W)])

    return _sc_gather


def _mm_body(ctx_ref, emb_ref, w_ref, b_ref, out_ref):
    combined = jnp.concatenate([ctx_ref[...], emb_ref[...]], axis=1)
    acc = lax.dot_general(
        combined, w_ref[...],
        (((1,), (1,)), ((), ())),
        preferred_element_type=jnp.float32,
    )
    out_ref[...] = acc + b_ref[...]


_BLK_N = 512


@jax.jit
def kernel(jrnl_variable, context_vectors, emb_table, W, b):
    idx = jrnl_variable.reshape((BATCH,)).astype(jnp.int32)
    emb = _get_sc_gather()(idx, emb_table)

    n_blocks = pl.cdiv(MESH_SIZE, _BLK_N)
    b2d = b.reshape((1, MESH_SIZE))
    out = pl.pallas_call(
        _mm_body,
        grid=(n_blocks,),
        in_specs=[
            pl.BlockSpec((BATCH, CTX_DIM), lambda i: (0, 0)),
            pl.BlockSpec((BATCH, JRNL_DIM), lambda i: (0, 0)),
            pl.BlockSpec((_BLK_N, IN_FEAT), lambda i: (i, 0)),
            pl.BlockSpec((1, _BLK_N), lambda i: (0, i)),
        ],
        out_specs=pl.BlockSpec((BATCH, _BLK_N), lambda i: (0, i)),
        out_shape=jax.ShapeDtypeStruct((BATCH, MESH_SIZE), jnp.float32),
        compiler_params=pltpu.CompilerParams(
            dimension_semantics=("arbitrary",),
        ),
    )(context_vectors, emb, W, b2d)
    return out


# speedup vs baseline: 2.8688x; 2.8688x over previous
import functools

import jax
import jax.numpy as jnp
from jax import lax
from jax.experimental import pallas as pl
from jax.experimental.pallas import tpu as pltpu
from jax.experimental.pallas import tpu_sc as plsc

N_JRNL = 1000000
JRNL_DIM = 32
MESH_SIZE = 28340
HIDDEN_SIZE = 128
N_PROBES = 3
BATCH = 1024
CTX_DIM = HIDDEN_SIZE * N_PROBES
IN_FEAT = CTX_DIM + JRNL_DIM

_NC = 2
_NS = 16
_NW = _NC * _NS
_B_PER_W = BATCH // _NW


@functools.cache
def _get_sc_gather():
    @functools.partial(
        pl.kernel,
        out_type=jax.ShapeDtypeStruct((BATCH, JRNL_DIM), jnp.float32),
        mesh=plsc.VectorSubcoreMesh(
            core_axis_name="c", subcore_axis_name="s",
            num_cores=_NC, num_subcores=_NS),
        scratch_types=[
            pltpu.VMEM((_B_PER_W,), jnp.int32),
            pltpu.VMEM((_B_PER_W, JRNL_DIM), jnp.float32),
            pltpu.SemaphoreType.DMA,
        ],
        compiler_params=pltpu.CompilerParams(use_tc_tiling_on_sc=False),
    )
    def _sc_gather(idx_hbm, table_hbm, out_hbm, idx_v, rows_v, sem):
        wid = lax.axis_index("s") * _NC + lax.axis_index("c")
        base = wid * _B_PER_W
        pltpu.sync_copy(idx_hbm.at[pl.ds(base, _B_PER_W)], idx_v)
        pltpu.async_copy(table_hbm.at[idx_v], rows_v, sem).wait()
        pltpu.sync_copy(rows_v, out_hbm.at[pl.ds(base, _B_PER_W)])

    return _sc_gather


def _mm_body(ctx_ref, emb_ref, w_ref, b_ref, out_ref):
    combined = jnp.concatenate([ctx_ref[...], emb_ref[...]], axis=1)
    acc = lax.dot_general(
        combined, w_ref[...],
        (((1,), (1,)), ((), ())),
        preferred_element_type=jnp.float32,
    )
    out_ref[...] = acc + b_ref[...]


_BLK_N = 512


@jax.jit
def kernel(jrnl_variable, context_vectors, emb_table, W, b):
    idx = jrnl_variable.reshape((BATCH,)).astype(jnp.int32)
    emb = jnp.take(emb_table, idx, axis=0)

    n_blocks = pl.cdiv(MESH_SIZE, _BLK_N)
    b2d = b.reshape((1, MESH_SIZE))
    out = pl.pallas_call(
        _mm_body,
        grid=(n_blocks,),
        in_specs=[
            pl.BlockSpec((BATCH, CTX_DIM), lambda i: (0, 0)),
            pl.BlockSpec((BATCH, JRNL_DIM), lambda i: (0, 0)),
            pl.BlockSpec((_BLK_N, IN_FEAT), lambda i: (i, 0)),
            pl.BlockSpec((1, _BLK_N), lambda i: (0, i)),
        ],
        out_specs=pl.BlockSpec((BATCH, _BLK_N), lambda i: (0, i)),
        out_shape=jax.ShapeDtypeStruct((BATCH, MESH_SIZE), jnp.float32),
        compiler_params=pltpu.CompilerParams(
            dimension_semantics=("arbitrary",),
        ),
    )(context_vectors, emb, W, b2d)
    return out


# speedup vs baseline: 3.1908x; 1.1122x over previous
import functools

import jax
import jax.numpy as jnp
from jax import lax
from jax.experimental import pallas as pl
from jax.experimental.pallas import tpu as pltpu
from jax.experimental.pallas import tpu_sc as plsc

N_JRNL = 1000000
JRNL_DIM = 32
MESH_SIZE = 28340
HIDDEN_SIZE = 128
N_PROBES = 3
BATCH = 1024
CTX_DIM = HIDDEN_SIZE * N_PROBES
IN_FEAT = CTX_DIM + JRNL_DIM

_NC = 2
_NS = 16
_NW = _NC * _NS
_B_PER_W = BATCH // _NW


@functools.cache
def _get_sc_gather():
    @functools.partial(
        pl.kernel,
        out_type=jax.ShapeDtypeStruct((BATCH, JRNL_DIM), jnp.float32),
        mesh=plsc.VectorSubcoreMesh(
            core_axis_name="c", subcore_axis_name="s",
            num_cores=_NC, num_subcores=_NS),
        scratch_types=[
            pltpu.VMEM((_B_PER_W,), jnp.int32),
            pltpu.VMEM((_B_PER_W, JRNL_DIM), jnp.float32),
            pltpu.SemaphoreType.DMA,
        ],
        compiler_params=pltpu.CompilerParams(use_tc_tiling_on_sc=False),
    )
    def _sc_gather(idx_hbm, table_hbm, out_hbm, idx_v, rows_v, sem):
        wid = lax.axis_index("s") * _NC + lax.axis_index("c")
        base = wid * _B_PER_W
        pltpu.sync_copy(idx_hbm.at[pl.ds(base, _B_PER_W)], idx_v)
        pltpu.async_copy(table_hbm.at[idx_v], rows_v, sem).wait()
        pltpu.sync_copy(rows_v, out_hbm.at[pl.ds(base, _B_PER_W)])

    return _sc_gather


def _mm_body(ctx_ref, emb_ref, w_ref, b_ref, out_ref):
    combined = jnp.concatenate([ctx_ref[...], emb_ref[...]], axis=1)
    acc = lax.dot_general(
        combined, w_ref[...],
        (((1,), (1,)), ((), ())),
        preferred_element_type=jnp.float32,
    )
    out_ref[...] = acc + b_ref[...]


_BLK_N = 2048


@jax.jit
def kernel(jrnl_variable, context_vectors, emb_table, W, b):
    idx = jrnl_variable.reshape((BATCH,)).astype(jnp.int32)
    emb = jnp.take(emb_table, idx, axis=0)

    n_blocks = pl.cdiv(MESH_SIZE, _BLK_N)
    b2d = b.reshape((1, MESH_SIZE))
    out = pl.pallas_call(
        _mm_body,
        grid=(n_blocks,),
        in_specs=[
            pl.BlockSpec((BATCH, CTX_DIM), lambda i: (0, 0)),
            pl.BlockSpec((BATCH, JRNL_DIM), lambda i: (0, 0)),
            pl.BlockSpec((_BLK_N, IN_FEAT), lambda i: (i, 0)),
            pl.BlockSpec((1, _BLK_N), lambda i: (0, i)),
        ],
        out_specs=pl.BlockSpec((BATCH, _BLK_N), lambda i: (0, i)),
        out_shape=jax.ShapeDtypeStruct((BATCH, MESH_SIZE), jnp.float32),
        compiler_params=pltpu.CompilerParams(
            dimension_semantics=("arbitrary",),
        ),
    )(context_vectors, emb, W, b2d)
    return out


# speedup vs baseline: 7.7059x; 2.4151x over previous
import functools

import jax
import jax.numpy as jnp
from jax import lax
from jax.experimental import pallas as pl
from jax.experimental.pallas import tpu as pltpu
from jax.experimental.pallas import tpu_sc as plsc

N_JRNL = 1000000
JRNL_DIM = 32
MESH_SIZE = 28340
HIDDEN_SIZE = 128
N_PROBES = 3
BATCH = 1024
CTX_DIM = HIDDEN_SIZE * N_PROBES
IN_FEAT = CTX_DIM + JRNL_DIM

_BLK_N = 2048


def _mm_body(wt_ref, comb_ref, b_ref, out_ref):
    acc = lax.dot_general(
        wt_ref[...], comb_ref[...],
        (((0,), (0,)), ((), ())),
        preferred_element_type=jnp.float32,
    )
    out_ref[...] = acc + b_ref[...]


@jax.jit
def kernel(jrnl_variable, context_vectors, emb_table, W, b):
    idx = jrnl_variable.reshape((BATCH,))
    emb_t = jnp.take(emb_table, idx, axis=0).T
    ctx_t = context_vectors.T
    comb_t = jnp.concatenate([ctx_t, emb_t], axis=0)
    wt = W.T
    b2d = b.reshape((MESH_SIZE, 1))

    n_blocks = pl.cdiv(MESH_SIZE, _BLK_N)
    out_t = pl.pallas_call(
        _mm_body,
        grid=(n_blocks,),
        in_specs=[
            pl.BlockSpec((IN_FEAT, _BLK_N), lambda i: (0, i)),
            pl.BlockSpec((IN_FEAT, BATCH), lambda i: (0, 0)),
            pl.BlockSpec((_BLK_N, 1), lambda i: (i, 0)),
        ],
        out_specs=pl.BlockSpec((_BLK_N, BATCH), lambda i: (i, 0)),
        out_shape=jax.ShapeDtypeStruct((MESH_SIZE, BATCH), jnp.float32),
        compiler_params=pltpu.CompilerParams(
            dimension_semantics=("arbitrary",),
        ),
    )(wt, comb_t, b2d)
    return out_t.T
